# baseline (speedup 1.0000x reference)
import functools

import jax
import jax.numpy as jnp
from jax import lax
from jax.experimental import pallas as pl
from jax.experimental.pallas import tpu as pltpu
from jax.experimental.pallas import tpu_sc as plsc

TOPK_N = 16
NEG = -3.0e38
BIG = 1e9
N_COLS = 8192
N_GROUPS = 512
N_SLABS = 16


def _select_groups_kernel(x_ref, wg_ref):
    x = x_ref[...]
    rows = x.shape[0]
    gm = x[:, 0:N_GROUPS]
    for s in range(1, N_SLABS):
        gm = jnp.maximum(gm, x[:, s * N_GROUPS:(s + 1) * N_GROUPS])
    iota = lax.broadcasted_iota(jnp.int32, (rows, N_GROUPS), 1).astype(jnp.float32)
    js = []
    for _ in range(TOPK_N):
        m = jnp.max(gm, axis=1, keepdims=True)
        cand = jnp.where(gm == m, iota, BIG)
        j = jnp.min(cand, axis=1, keepdims=True)
        js.append(j)
        gm = jnp.where(iota == j, NEG, gm)
    wg_ref[...] = jnp.concatenate(js, axis=1).astype(jnp.int32)


def _vsqrt(a):
    ai = plsc.bitcast(a, jnp.int32)
    y = plsc.bitcast((ai >> 1) + 0x1FBD1DF5, jnp.float32)
    for _ in range(3):
        y = 0.5 * (y + a / y)
    return y


def _bvec(s):
    return lax.broadcast_in_dim(s, (16,), ())


def _lex_merge(a, b):
    av, ai = a
    rb = lax.rev(b[0], (0,))
    rbi = lax.rev(b[1], (0,))
    take = (av > rb) | ((av == rb) & (ai < rbi))
    lv = jnp.where(take, av, rb)
    li = jnp.where(take, ai, rbi)
    return plsc.sort_key_val(lv, li, descending=True)


def _tie_fix_pass(rv, ri, lane, perm):
    _, pv = plsc.sort_key_val(perm, rv)
    _, pi = plsc.sort_key_val(perm, ri)
    eq = rv == pv
    first = lane < perm
    return jnp.where(eq, jnp.where(first, jnp.minimum(ri, pi),
                                   jnp.maximum(ri, pi)), ri)


def _sc_row(r, xbuf, wgv, outv, lane, off=0):
    wrow = wgv[r, :]
    level = []
    for s in range(N_SLABS):
        idxv = wrow + s * N_GROUPS
        c = plsc.load_gather(xbuf, [idxv + off])
        level.append(plsc.sort_key_val(c, idxv, descending=True))
    while len(level) > 1:
        level = [_lex_merge(level[i], level[i + 1])
                 for i in range(0, len(level), 2)]
    rv, ri = level[0]
    odd = (lane & 1) == 1
    perm1 = lane ^ 1
    perm2 = jnp.where((lane >= 1) & (lane <= 14),
                      jnp.where(odd, lane + 1, lane - 1), lane)
    ri = _tie_fix_pass(rv, ri, lane, perm1)
    ri = _tie_fix_pass(rv, ri, lane, perm2)
    ri = _tie_fix_pass(rv, ri, lane, perm1)
    rif = ri.astype(jnp.float32)
    m3 = lane < 3
    m0 = lane == 0
    top3_mean = _bvec(jnp.sum(jnp.where(m3, rif, 0.0))) * (1.0 / 3.0)
    top3_rms = _vsqrt(_bvec(jnp.sum(jnp.where(m3, rv * rv, 0.0))) * (1.0 / 3.0))
    j0 = _bvec(jnp.sum(jnp.where(m0, rif, 0.0)))
    max_rms = jnp.abs(_bvec(jnp.sum(jnp.where(m0, rv, 0.0))))
    sv = jnp.where(lane == 0, top3_mean,
                   jnp.where(lane == 1, top3_rms,
                             jnp.where(lane == 2, j0, max_rms)))
    outv[r, pl.ds(0, 16)] = sv
    outv[r, pl.ds(16, 16)] = rif


def _sc_topk(x, wg, n_rows):
    try:
        info = plsc.get_sparse_core_info()
        nc, ns = info.num_cores, info.num_subcores
    except Exception:
        nc, ns = 2, 16
    nw = nc * ns
    rows_per_w = n_rows // nw
    n_pairs = rows_per_w // 2
    mesh = plsc.VectorSubcoreMesh(
        core_axis_name="c", subcore_axis_name="s",
        num_cores=nc, num_subcores=ns)

    rows_per_b = 4
    batch_elems = rows_per_b * N_COLS
    n_bpairs = rows_per_w // (2 * rows_per_b)
    del n_pairs

    @functools.partial(
        pl.kernel,
        mesh=mesh,
        out_type=jax.ShapeDtypeStruct((n_rows, 32), jnp.float32),
        compiler_params=pltpu.CompilerParams(needs_layout_passes=False),
        scratch_types=[
            pltpu.VMEM((batch_elems,), jnp.float32),
            pltpu.VMEM((batch_elems,), jnp.float32),
            pltpu.VMEM((rows_per_w, TOPK_N), jnp.int32),
            pltpu.VMEM((rows_per_w, 32), jnp.float32),
            pltpu.SemaphoreType.DMA,
            pltpu.SemaphoreType.DMA,
        ],
    )
    def sc_kernel(xf_hbm, wg_hbm, out_hbm, xbuf0, xbuf1, wgv, outv,
                  sem0, sem1):
        wid = lax.axis_index("s") * nc + lax.axis_index("c")
        base = wid * rows_per_w
        ebase = base * N_COLS
        lane = lax.iota(jnp.int32, 16)
        pltpu.sync_copy(wg_hbm.at[pl.ds(base, rows_per_w), :], wgv)
        pltpu.async_copy(xf_hbm.at[pl.ds(ebase, batch_elems)], xbuf0, sem0)

        def bpair_body(p, carry):
            b0 = 2 * p
            e0 = ebase + b0 * batch_elems
            pltpu.async_copy(
                xf_hbm.at[pl.ds(e0 + batch_elems, batch_elems)], xbuf1, sem1)
            pltpu.make_async_copy(
                xf_hbm.at[pl.ds(e0, batch_elems)], xbuf0, sem0).wait()
            for k in range(rows_per_b):
                _sc_row(b0 * rows_per_b + k, xbuf0, wgv, outv, lane,
                        off=k * N_COLS)

            @pl.when(p < n_bpairs - 1)
            def _():
                pltpu.async_copy(
                    xf_hbm.at[pl.ds(e0 + 2 * batch_elems, batch_elems)],
                    xbuf0, sem0)

            pltpu.make_async_copy(
                xf_hbm.at[pl.ds(e0, batch_elems)], xbuf1, sem1).wait()
            for k in range(rows_per_b):
                _sc_row((b0 + 1) * rows_per_b + k, xbuf1, wgv, outv, lane,
                        off=k * N_COLS)
            return carry

        lax.fori_loop(0, n_bpairs, bpair_body, 0)
        pltpu.sync_copy(outv, out_hbm.at[pl.ds(base, rows_per_w), :])

    return sc_kernel(x.reshape(-1), wg)


def kernel(inputs):
    n_rows = inputs.shape[1] // 2
    x = inputs[:n_rows]
    block_rows = 256
    wg = pl.pallas_call(
        _select_groups_kernel,
        grid=(n_rows // block_rows,),
        in_specs=[pl.BlockSpec((block_rows, N_COLS), lambda i: (i, 0))],
        out_specs=pl.BlockSpec((block_rows, TOPK_N), lambda i: (i, 0)),
        out_shape=jax.ShapeDtypeStruct((n_rows, TOPK_N), jnp.int32),
    )(x)
    out32 = _sc_topk(x, wg, n_rows)
    return jnp.concatenate([out32[:, :4], out32[:, 16:32]], axis=1)


# speedup vs baseline: 1.2998x; 1.2998x over previous
import functools

import jax
import jax.numpy as jnp
from jax import lax
from jax.experimental import pallas as pl
from jax.experimental.pallas import tpu as pltpu
from jax.experimental.pallas import tpu_sc as plsc

TOPK_N = 16
NEG = -3.0e38
BIG = 1e9
N_COLS = 8192
N_GROUPS = 512
N_SLABS = 16


def _select_groups_kernel(x_ref, wg_ref):
    x = x_ref[...]
    rows = x.shape[0]
    iota = lax.broadcasted_iota(jnp.int32, (rows, N_GROUPS), 1).astype(jnp.float32)
    gm = x[:, 0:N_GROUPS]
    ei = iota
    for s in range(1, N_SLABS):
        slab = x[:, s * N_GROUPS:(s + 1) * N_GROUPS]
        better = slab > gm
        gm = jnp.where(better, slab, gm)
        ei = jnp.where(better, iota + (s * N_GROUPS), ei)
    js = []
    for _ in range(TOPK_N):
        m = jnp.max(gm, axis=1, keepdims=True)
        cand = jnp.where(gm == m, ei, BIG)
        j = jnp.min(cand, axis=1, keepdims=True)
        js.append(j)
        gm = jnp.where(cand == j, NEG, gm)
    wg_ref[...] = jnp.concatenate(js, axis=1).astype(jnp.int32) & (N_GROUPS - 1)


def _vsqrt(a):
    ai = plsc.bitcast(a, jnp.int32)
    y = plsc.bitcast((ai >> 1) + 0x1FBD1DF5, jnp.float32)
    for _ in range(3):
        y = 0.5 * (y + a / y)
    return y


def _bvec(s):
    return lax.broadcast_in_dim(s, (16,), ())


def _merge(a, b):
    av, ai = a
    rb = lax.rev(b[0], (0,))
    rbi = lax.rev(b[1], (0,))
    take = av >= rb
    lv = jnp.where(take, av, rb)
    li = jnp.where(take, ai, rbi)
    hv = jnp.where(take, rb, av)
    sk, svv = plsc.sort_key_val(lv, li, descending=True)
    return (sk, svv), hv


def _stats_store(r, rv, rif, lane, outv):
    m3 = lane < 3
    m0 = lane == 0
    top3_mean = _bvec(jnp.sum(jnp.where(m3, rif, 0.0))) * (1.0 / 3.0)
    top3_rms = _vsqrt(_bvec(jnp.sum(jnp.where(m3, rv * rv, 0.0))) * (1.0 / 3.0))
    j0 = _bvec(jnp.sum(jnp.where(m0, rif, 0.0)))
    max_rms = jnp.abs(_bvec(jnp.sum(jnp.where(m0, rv, 0.0))))
    sv = jnp.where(lane == 0, top3_mean,
                   jnp.where(lane == 1, top3_rms,
                             jnp.where(lane == 2, j0, max_rms)))
    outv[r, pl.ds(0, 16)] = sv
    outv[r, pl.ds(16, 16)] = rif


def _sc_row(r, xbuf, wgv, outv, lane):
    wrow = wgv[r, :]
    chunks = []
    for s in range(N_SLABS):
        idxv = wrow + s * N_GROUPS
        c = plsc.load_gather(xbuf, [idxv])
        chunks.append((c, idxv))
    level = [plsc.sort_key_val(c, i, descending=True) for c, i in chunks]
    hmax = None
    while len(level) > 1:
        nxt = []
        for i in range(0, len(level), 2):
            m, hv = _merge(level[i], level[i + 1])
            nxt.append(m)
            hmax = hv if hmax is None else jnp.maximum(hmax, hv)
        level = nxt
    rv, ri = level[0]
    odd = (lane & 1) == 1
    perm1 = lane ^ 1
    perm2 = jnp.where((lane >= 1) & (lane <= 14),
                      jnp.where(odd, lane + 1, lane - 1), lane)
    _, p1 = plsc.sort_key_val(perm1, rv)
    _, p2 = plsc.sort_key_val(perm2, rv)
    adj = ((rv == p1) | ((rv == p2) & (perm2 != lane)))
    n_adj = jnp.max(plsc.all_reduce_population_count(adj))
    minr = jnp.min(rv)
    risk = (n_adj > 0) | (jnp.max(hmax) >= minr)

    @pl.when(jnp.logical_not(risk))
    def _():
        _stats_store(r, rv, ri.astype(jnp.float32), lane, outv)

    @pl.when(risk)
    def _():
        lcs = [c for c, _ in chunks]
        ivs = [iv for _, iv in chunks]
        vals = lane.astype(jnp.float32) * 0.0
        idxf = vals
        for t in range(TOPK_N):
            m = lcs[0]
            for c in lcs[1:]:
                m = jnp.maximum(m, c)
            vmax = _bvec(jnp.max(m))
            jm = None
            for c, iv in zip(lcs, ivs):
                cand = jnp.where(c == vmax, iv, 2 ** 30)
                jm = cand if jm is None else jnp.minimum(jm, cand)
            jmin = _bvec(jnp.min(jm))
            sel = lane == t
            vals = jnp.where(sel, vmax, vals)
            idxf = jnp.where(sel, jmin.astype(jnp.float32), idxf)
            lcs = [jnp.where(iv == jmin, NEG, c)
                   for c, iv in zip(lcs, ivs)]
        _stats_store(r, vals, idxf, lane, outv)


def _sc_topk(x, wg, n_rows):
    try:
        info = plsc.get_sparse_core_info()
        nc, ns = info.num_cores, info.num_subcores
    except Exception:
        nc, ns = 2, 16
    nw = nc * ns
    rows_per_w = n_rows // nw
    n_pairs = rows_per_w // 2
    mesh = plsc.VectorSubcoreMesh(
        core_axis_name="c", subcore_axis_name="s",
        num_cores=nc, num_subcores=ns)

    @functools.partial(
        pl.kernel,
        mesh=mesh,
        out_type=jax.ShapeDtypeStruct((n_rows, 32), jnp.float32),
        compiler_params=pltpu.CompilerParams(needs_layout_passes=False),
        scratch_types=[
            pltpu.VMEM((N_COLS,), jnp.float32),
            pltpu.VMEM((N_COLS,), jnp.float32),
            pltpu.VMEM((rows_per_w, TOPK_N), jnp.int32),
            pltpu.VMEM((rows_per_w, 32), jnp.float32),
            pltpu.SemaphoreType.DMA,
            pltpu.SemaphoreType.DMA,
        ],
    )
    def sc_kernel(x_hbm, wg_hbm, out_hbm, xbuf0, xbuf1, wgv, outv,
                  sem0, sem1):
        wid = lax.axis_index("s") * nc + lax.axis_index("c")
        base = wid * rows_per_w
        lane = lax.iota(jnp.int32, 16)
        pltpu.sync_copy(wg_hbm.at[pl.ds(base, rows_per_w), :], wgv)
        pltpu.async_copy(x_hbm.at[base], xbuf0, sem0)

        def pair_body(p, carry):
            r0 = 2 * p
            pltpu.async_copy(x_hbm.at[base + r0 + 1], xbuf1, sem1)
            pltpu.make_async_copy(x_hbm.at[base + r0], xbuf0, sem0).wait()
            _sc_row(r0, xbuf0, wgv, outv, lane)

            @pl.when(p < n_pairs - 1)
            def _():
                pltpu.async_copy(x_hbm.at[base + r0 + 2], xbuf0, sem0)

            pltpu.make_async_copy(x_hbm.at[base + r0 + 1], xbuf1, sem1).wait()
            _sc_row(r0 + 1, xbuf1, wgv, outv, lane)
            return carry

        lax.fori_loop(0, n_pairs, pair_body, 0)
        pltpu.sync_copy(outv, out_hbm.at[pl.ds(base, rows_per_w), :])

    return sc_kernel(x, wg)


def kernel(inputs):
    n_rows = inputs.shape[1] // 2
    x = inputs[:n_rows]
    block_rows = 256
    wg = pl.pallas_call(
        _select_groups_kernel,
        grid=(n_rows // block_rows,),
        in_specs=[pl.BlockSpec((block_rows, N_COLS), lambda i: (i, 0))],
        out_specs=pl.BlockSpec((block_rows, TOPK_N), lambda i: (i, 0)),
        out_shape=jax.ShapeDtypeStruct((n_rows, TOPK_N), jnp.int32),
    )(x)
    out32 = _sc_topk(x, wg, n_rows)
    return jnp.concatenate([out32[:, :4], out32[:, 16:32]], axis=1)
